# Initial kernel scaffold; baseline (speedup 1.0000x reference)
import jax
import jax.numpy as jnp
from jax.experimental import pallas as pl


def kernel(x_paper, x_author, edge_index_cites, edge_index_writes, edge_index_rev, batch_paper, batch_author, y_base, Win, b_in, ln_g, ln_b, Wk, bk, Wq, bq, Wv, bv, a_rel, m_rel, p_rel, skip, Wo, bo, Wjk, bjk, Wy1, by1, Wy2, by2, G1, gb1, bn1g, bn1b, G2, gb2, bn2g, bn2b, G3, gb3, bn3g, bn3b, G4, gb4):
    raise NotImplementedError("write your pallas kernel here")


# speedup vs baseline: 15.3341x; 15.3341x over previous
import functools
import math

import jax
import jax.numpy as jnp
from jax import lax
from jax.experimental import pallas as pl
from jax.experimental.pallas import tpu as pltpu
from jax.experimental.pallas import tpu_sc as plsc

NP_, NA_ = 30000, 20000
HID = 128
H = 4
DH = 32
L = 2
T = 2
ETS = [(0, 0), (1, 0), (0, 1)]
SIZES = [NP_, NA_]
NGRAPH = 64

BLK = 256
NPP = 30208
NAP = 20224
PAD_SIZES = [NPP, NAP]
CHUNK = 128
NTILES = 16
EC_P = 303104
EW_P = 151552


def _gelu(x):
    return 0.5 * x * (1.0 + lax.erf(x * (1.0 / math.sqrt(2.0))))


def _pre_body(nreal, nkv, layer1, *refs):
    if layer1:
        (x_ref, win_ref, bin_ref, lng_ref, lnb_ref, wt_ref, bt_ref,
         xn_ref, q_ref, *outs) = refs
    else:
        (x_ref, lng_ref, lnb_ref, wt_ref, bt_ref, xn_ref, q_ref,
         *outs) = refs
    i = pl.program_id(0)
    rows = lax.broadcasted_iota(jnp.int32, (BLK, 1), 0) + i * BLK
    mask = rows < nreal
    x = x_ref[...]
    if layer1:
        x = x @ win_ref[...] + bin_ref[...]
    m = jnp.mean(x, axis=-1, keepdims=True)
    v = jnp.mean((x - m) ** 2, axis=-1, keepdims=True)
    xn = (x - m) * lax.rsqrt(v + 1e-5) * lng_ref[...] + lnb_ref[...]
    xn = jnp.where(mask, xn, 0.0)
    xn_ref[...] = xn
    y = xn @ wt_ref[...] + bt_ref[...]
    y = jnp.where(mask, y, 0.0)
    for h in range(H):
        q_ref[h, :, :] = y[:, h * DH:(h + 1) * DH]
    for o in range(nkv):
        kbase = HID * (1 + 2 * o)
        vbase = HID * (2 + 2 * o)
        for h in range(H):
            outs[o][h, :, 0:DH] = y[:, kbase + h * DH:kbase + (h + 1) * DH]
            outs[o][h, :, DH:2 * DH] = y[:, vbase + h * DH:
                                          vbase + (h + 1) * DH]


def _pre_call(x, win, bin_, lng, lnb, wt, bt, npad, nreal, nkv, layer1):
    nblk = npad // BLK
    wcols = (1 + 2 * nkv) * HID
    row_spec = pl.BlockSpec((BLK, HID), lambda i: (i, 0))
    full = lambda shape: pl.BlockSpec(shape, lambda i: tuple(0 for _ in shape))
    in_specs = [row_spec]
    args = [x]
    if layer1:
        in_specs += [full((HID, HID)), full((1, HID))]
        args += [win, bin_]
    in_specs += [full((1, HID)), full((1, HID)), full((HID, wcols)),
                 full((1, wcols))]
    args += [lng.reshape(1, HID), lnb.reshape(1, HID), wt,
             bt.reshape(1, wcols)]
    out_shapes = ([jax.ShapeDtypeStruct((npad, HID), jnp.float32),
                   jax.ShapeDtypeStruct((H, npad, DH), jnp.float32)] +
                  [jax.ShapeDtypeStruct((H, npad, 2 * DH), jnp.float32)
                   for _ in range(nkv)])
    out_specs = ([row_spec,
                  pl.BlockSpec((H, BLK, DH), lambda i: (0, i, 0))] +
                 [pl.BlockSpec((H, BLK, 2 * DH), lambda i: (0, i, 0))] * nkv)
    return pl.pallas_call(
        functools.partial(_pre_body, nreal, nkv, layer1),
        grid=(nblk,),
        in_specs=in_specs,
        out_specs=out_specs,
        out_shape=out_shapes,
    )(*args)


def _post_body(nreal, nsrcs, *refs):
    aggs = refs[0:nsrcs]
    xn_ref, wo_ref, bo_ref, beta_ref, out_ref = refs[nsrcs:]
    i = pl.program_id(0)
    rows = lax.broadcasted_iota(jnp.int32, (BLK, 1), 0) + i * BLK
    mask = rows < nreal
    parts = []
    for h in range(H):
        acc = None
        for a_ref in aggs:
            a = a_ref[h, :, 0:DH]
            d = a_ref[h, :, DH:DH + 1]
            term = a / (d + 1e-16)
            acc = term if acc is None else acc + term
        parts.append(acc)
    full = jnp.concatenate(parts, axis=1)
    o = _gelu(full) @ wo_ref[...] + bo_ref[...]
    beta = beta_ref[0, 0]
    xnew = beta * o + (1.0 - beta) * xn_ref[...]
    out_ref[...] = jnp.where(mask, xnew, 0.0)


def _post_call(aggdens, xn, wo, bo, beta, npad, nreal):
    nblk = npad // BLK
    row_spec = pl.BlockSpec((BLK, HID), lambda i: (i, 0))
    full = lambda shape: pl.BlockSpec(shape, lambda i: tuple(0 for _ in shape))
    in_specs = []
    args = []
    for a in aggdens:
        in_specs += [pl.BlockSpec((H, BLK, AGGW), lambda i: (0, i, 0))]
        args += [a]
    in_specs += [row_spec, full((HID, HID)), full((1, HID)), full((1, 1))]
    args += [xn, wo, bo.reshape(1, HID), beta.reshape(1, 1)]
    return pl.pallas_call(
        functools.partial(_post_body, nreal, len(aggdens)),
        grid=(nblk,),
        in_specs=in_specs,
        out_specs=row_spec,
        out_shape=jax.ShapeDtypeStruct((npad, HID), jnp.float32),
    )(*args)


def _pool_body(nblk, x1_ref, x2_ref, b_ref, bc_ref, w1_ref, w2_ref, bj_ref,
               add_ref, max_ref):
    i = pl.program_id(0)

    @pl.when(i == 0)
    def _():
        add_ref[...] = jnp.zeros_like(add_ref)
        max_ref[...] = jnp.full_like(max_ref, -1e30)

    xc = x1_ref[...] @ w1_ref[...] + x2_ref[...] @ w2_ref[...] + bj_ref[...]
    batch = b_ref[0]
    gids = lax.broadcasted_iota(jnp.int32, (NGRAPH, 1), 0)
    oht = (gids == batch).astype(jnp.float32)
    add_ref[...] += lax.dot_general(oht, xc, (((1,), (0,)), ((), ())),
                                    preferred_element_type=jnp.float32)
    bcol = bc_ref[...]

    def gloop(g, _):
        mrow = jnp.max(jnp.where(bcol == g, xc, -1e30), axis=0,
                       keepdims=True)
        max_ref[pl.ds(g, 1), :] = jnp.maximum(max_ref[pl.ds(g, 1), :], mrow)
        return 0
    lax.fori_loop(0, NGRAPH, gloop, 0)

    @pl.when(i == nblk - 1)
    def _():
        mx = max_ref[...]
        max_ref[...] = jnp.where(mx <= -1e29, 0.0, mx)


def _pool_call(x1, x2, batch3, batch2, w1, w2, bj, npad):
    nblk = npad // BLK
    row_spec = pl.BlockSpec((BLK, HID), lambda i: (i, 0))
    full = lambda shape: pl.BlockSpec(shape, lambda i: tuple(0 for _ in shape))
    out_spec = pl.BlockSpec((NGRAPH, HID), lambda i: (0, 0))
    return pl.pallas_call(
        functools.partial(_pool_body, nblk),
        grid=(nblk,),
        in_specs=[row_spec, row_spec,
                  pl.BlockSpec((1, 1, BLK), lambda i: (i, 0, 0)),
                  pl.BlockSpec((BLK, 1), lambda i: (i, 0)),
                  full((HID, HID)), full((HID, HID)), full((1, HID))],
        out_specs=[out_spec, out_spec],
        out_shape=[jax.ShapeDtypeStruct((NGRAPH, HID), jnp.float32),
                   jax.ShapeDtypeStruct((NGRAPH, HID), jnp.float32)],
    )(x1, x2, batch3, batch2, w1, w2, bj.reshape(1, HID))


def _head_body(ap_ref, mp_ref, aa_ref, ma_ref, yb_ref, wy1_ref, by1_ref,
               wy2_ref, by2_ref, G1_ref, gb1_ref, g1g_ref, g1b_ref,
               G2_ref, gb2_ref, g2g_ref, g2b_ref,
               G3_ref, gb3_ref, g3g_ref, g3b_ref,
               G4_ref, gb4_ref, out_ref):
    def bn(x, gg, bb):
        m = jnp.mean(x, axis=0, keepdims=True)
        vv = jnp.mean((x - m) ** 2, axis=0, keepdims=True)
        return (x - m) * lax.rsqrt(vv + 1e-5) * gg + bb

    y = yb_ref[...] @ wy1_ref[...] + by1_ref[...]
    y = jnp.where(y > 0, y, 0.2 * y)
    yb = y @ wy2_ref[...] + by2_ref[...]
    g = jnp.concatenate([ap_ref[...], mp_ref[...], aa_ref[...], ma_ref[...],
                         yb], axis=1)
    h = _gelu(bn(g @ G1_ref[...] + gb1_ref[...], g1g_ref[...], g1b_ref[...]))
    h = _gelu(bn(h @ G2_ref[...] + gb2_ref[...], g2g_ref[...], g2b_ref[...]))
    h = _gelu(bn(h @ G3_ref[...] + gb3_ref[...], g3g_ref[...], g3b_ref[...]))
    out_ref[...] = h @ G4_ref[...] + gb4_ref[...]


def _head_call(ap, mp, aa, ma, y_base, Wy1, by1, Wy2, by2, G1, gb1, bn1g,
               bn1b, G2, gb2, bn2g, bn2b, G3, gb3, bn3g, bn3b, G4, gb4):
    out = pl.pallas_call(
        _head_body,
        out_shape=jax.ShapeDtypeStruct((NGRAPH, 1), jnp.float32),
    )(ap, mp, aa, ma, y_base, Wy1, by1.reshape(1, 16), Wy2,
      by2.reshape(1, 16), G1, gb1.reshape(1, HID), bn1g.reshape(1, HID),
      bn1b.reshape(1, HID), G2, gb2.reshape(1, 128), bn2g.reshape(1, 128),
      bn2b.reshape(1, 128), G3, gb3.reshape(1, 64), bn3g.reshape(1, 64),
      bn3b.reshape(1, 64), G4, gb4.reshape(1, 1))
    return out[:, 0]


AGGW = 48


def _sc_body(epad, nsrc_pad, ndst_pad,
             kv, qq, src_h, dst_h, zrows, agg_h,
             idx_s, idx_d, kvrows, qrows, msg, agg_sh, sem):
    c = lax.axis_index("c")
    s = lax.axis_index("s")
    iota16 = lax.iota(jnp.int32, 16)
    zf16 = jnp.zeros((16,), jnp.float32)
    c32 = jnp.full((16,), DH, jnp.int32)

    def zero2d(ref, nrows, ncols):
        def zb(i, _):
            idx = i * 16 + iota16
            plsc.store_scatter(ref, [idx // ncols, idx % ncols], zf16)
            return 0
        lax.fori_loop(0, nrows * ncols // 16, zb, 0)

    zero2d(msg, CHUNK, AGGW)
    rpt = ndst_pad // NTILES
    nchunks = epad // NTILES // CHUNK
    tile_base = s * (epad // NTILES)

    def do_pass(p):
        hoff = (2 * c + p)

        pltpu.sync_copy(zrows.at[pl.ds(0, rpt)],
                        agg_sh.at[pl.ds(s * rpt, rpt)])
        plsc.subcore_barrier()

        def chunk_body(i, _):
            base = tile_base + i * CHUNK
            pltpu.sync_copy(src_h.at[pl.ds(base, CHUNK)], idx_s)
            pltpu.sync_copy(dst_h.at[pl.ds(base, CHUNK)], idx_d)

            def offs(j, _):
                sl = pl.ds(j * 16, 16)
                idx_s[sl] = idx_s[sl] + hoff * nsrc_pad
                idx_d[sl] = idx_d[sl] + hoff * ndst_pad
                return 0
            lax.fori_loop(0, CHUNK // 16, offs, 0)

            cp1 = pltpu.async_copy(kv.at[idx_s], kvrows, sem)
            cp2 = pltpu.async_copy(qq.at[idx_d], qrows, sem)
            cp1.wait()
            cp2.wait()

            def deoff(j, _):
                sl = pl.ds(j * 16, 16)
                idx_d[sl] = idx_d[sl] - hoff * ndst_pad
                return 0
            lax.fori_loop(0, CHUNK // 16, deoff, 0)

            def group_body(g, _):
                eids = iota16 + g * 16
                acc = jnp.zeros((16,), jnp.float32)
                for j in range(DH):
                    jj = jnp.full((16,), j, jnp.int32)
                    kj = plsc.load_gather(kvrows, [eids, jj])
                    qj = plsc.load_gather(qrows, [eids, jj])
                    acc = acc + kj * qj
                ee = jnp.exp(acc)
                plsc.store_scatter(msg, [eids, c32], ee)
                for e in range(16):
                    rowv = jnp.full((16,), g * 16 + e, jnp.int32)
                    b = ee.at[jnp.full((16,), e, jnp.int32)].get(
                        mode="promise_in_bounds")
                    for q2 in range(2):
                        cols = iota16 + q2 * 16
                        vvv = plsc.load_gather(kvrows, [rowv, cols + DH])
                        plsc.store_scatter(msg, [rowv, cols], vvv * b)
                return 0
            lax.fori_loop(0, CHUNK // 16, group_body, 0)

            pltpu.sync_copy(msg, agg_sh.at[idx_d], add=True)
            return 0
        lax.fori_loop(0, nchunks, chunk_body, 0)
        plsc.subcore_barrier()

        pltpu.sync_copy(agg_sh.at[pl.ds(s * rpt, rpt)],
                        agg_h.at[pl.ds(hoff * ndst_pad + s * rpt, rpt)])
        plsc.subcore_barrier()

    do_pass(0)
    do_pass(1)


def _sc_edge_type(kv, qq, src, dst, zrows, epad, nsrc_pad, ndst_pad):
    f32 = jnp.float32
    mesh = plsc.VectorSubcoreMesh(core_axis_name="c", subcore_axis_name="s",
                                  num_cores=2, num_subcores=NTILES)
    kern = pl.kernel(
        functools.partial(_sc_body, epad, nsrc_pad, ndst_pad),
        out_type=jax.ShapeDtypeStruct((H * ndst_pad, AGGW), f32),
        mesh=mesh,
        compiler_params=pltpu.CompilerParams(needs_layout_passes=False,
                                             use_tc_tiling_on_sc=False),
        scratch_types=[
            pltpu.VMEM((CHUNK,), jnp.int32),
            pltpu.VMEM((CHUNK,), jnp.int32),
            pltpu.VMEM((CHUNK, 2 * DH), f32),
            pltpu.VMEM((CHUNK, DH), f32),
            pltpu.VMEM((CHUNK, AGGW), f32),
            pltpu.VMEM_SHARED((NPP, AGGW), f32),
            pltpu.SemaphoreType.DMA,
        ],
    )
    return kern(kv, qq, src, dst, zrows)


def _sc_edge_pass(qp, qa, kvc, kvw, kvr,
                  src_c, dst_c, src_w, dst_w, src_r, dst_r, zrows):
    agg_c = _sc_edge_type(kvc, qp, src_c, dst_c, zrows, EC_P, NPP, NPP)
    agg_w = _sc_edge_type(kvw, qp, src_w, dst_w, zrows, EW_P, NAP, NPP)
    agg_r = _sc_edge_type(kvr, qa, src_r, dst_r, zrows, EW_P, NPP, NAP)
    return agg_c, agg_w, agg_r


def _pad_rows(x, npad):
    return jnp.pad(x, ((0, npad - x.shape[0]), (0, 0)))


def _pad_edges(src, dst, epad, dummy):
    e = src.shape[0]
    src = jnp.pad(src.astype(jnp.int32), (0, epad - e),
                  constant_values=dummy)
    dst = jnp.pad(dst.astype(jnp.int32), (0, epad - e),
                  constant_values=dummy)
    return src, dst


def _fuse_k(Wk_t, bk_t, rel, scale):
    w = Wk_t.reshape(HID, H, DH)
    w = jnp.einsum('nhd,hde->nhe', w, rel) * scale[None, :, None]
    b = bk_t.reshape(H, DH)
    b = jnp.einsum('hd,hde->he', b, rel) * scale[:, None]
    return w.reshape(HID, HID), b.reshape(HID)


def kernel(x_paper, x_author, edge_index_cites, edge_index_writes,
           edge_index_rev, batch_paper, batch_author, y_base, Win, b_in,
           ln_g, ln_b, Wk, bk, Wq, bq, Wv, bv, a_rel, m_rel, p_rel, skip,
           Wo, bo, Wjk, bjk, Wy1, by1, Wy2, by2, G1, gb1, bn1g, bn1b, G2,
           gb2, bn2g, bn2b, G3, gb3, bn3g, bn3b, G4, gb4):
    xp = _pad_rows(x_paper, NPP)
    xa = _pad_rows(x_author, NAP)
    src_c, dst_c = _pad_edges(edge_index_cites[0], edge_index_cites[1],
                              EC_P, NP_)
    src_w, dst_w = _pad_edges(edge_index_writes[0], edge_index_writes[1],
                              EW_P, NA_)
    dst_w = jnp.where(lax.iota(jnp.int32, EW_P) <
                      edge_index_writes.shape[1], dst_w, NP_)
    src_r, dst_r = _pad_edges(edge_index_rev[0], edge_index_rev[1],
                              EW_P, NP_)
    dst_r = jnp.where(lax.iota(jnp.int32, EW_P) <
                      edge_index_rev.shape[1], dst_r, NA_)

    scale = p_rel / math.sqrt(float(DH))

    xs = [xp, xa]
    jk = [[], []]
    for l in range(L):
        wts, bts, kouts = [], [], []
        for t in range(T):
            ws = [Wq[l, t]]
            bs = [bq[l, t]]
            for e, (st, dt) in enumerate(ETS):
                if st != t:
                    continue
                wkf, bkf = _fuse_k(Wk[l, t], bk[l, t], a_rel[l, e],
                                   scale[l, e])
                wvf, bvf = _fuse_k(Wv[l, t], bv[l, t], m_rel[l, e],
                                   jnp.ones((H,), jnp.float32))
                ws += [wkf, wvf]
                bs += [bkf, bvf]
            wts.append(jnp.concatenate(ws, axis=1))
            bts.append(jnp.concatenate(bs, axis=0))
            kouts.append(len(ws))

        pre_p = _pre_call(xs[0], Win[0], b_in[0].reshape(1, HID),
                          ln_g[l, 0], ln_b[l, 0], wts[0], bts[0],
                          NPP, NP_, 2, l == 0)
        pre_a = _pre_call(xs[1], Win[1], b_in[1].reshape(1, HID),
                          ln_g[l, 1], ln_b[l, 1], wts[1], bts[1],
                          NAP, NA_, 1, l == 0)
        xn_p, q_p, kvc3, kvr3 = pre_p
        xn_a, q_a, kvw3 = pre_a

        zrows = jnp.zeros((NPP // NTILES, AGGW), jnp.float32)
        agg_c, agg_w, agg_r = _sc_edge_pass(
            q_p.reshape(-1, DH), q_a.reshape(-1, DH),
            kvc3.reshape(-1, 2 * DH), kvw3.reshape(-1, 2 * DH),
            kvr3.reshape(-1, 2 * DH),
            src_c, dst_c, src_w, dst_w, src_r, dst_r, zrows)

        r3 = lambda a, npad: a.reshape(H, npad, -1)
        beta_p = jax.nn.sigmoid(skip[l, 0]).reshape(1, 1)
        beta_a = jax.nn.sigmoid(skip[l, 1]).reshape(1, 1)
        x_p = _post_call([r3(agg_c, NPP), r3(agg_w, NPP)],
                         xn_p, Wo[l, 0], bo[l, 0], beta_p, NPP, NP_)
        x_a = _post_call([r3(agg_r, NAP)],
                         xn_a, Wo[l, 1], bo[l, 1], beta_a, NAP, NA_)
        xs = [x_p, x_a]
        jk[0].append(x_p)
        jk[1].append(x_a)

    batch3 = [
        jnp.pad(batch_paper.astype(jnp.int32), (0, NPP - NP_),
                constant_values=NGRAPH).reshape(NPP // BLK, 1, BLK),
        jnp.pad(batch_author.astype(jnp.int32), (0, NAP - NA_),
                constant_values=NGRAPH).reshape(NAP // BLK, 1, BLK),
    ]
    pools = []
    for t in range(T):
        add_p, max_p = _pool_call(jk[t][0], jk[t][1], batch3[t],
                                  batch3[t].reshape(PAD_SIZES[t], 1),
                                  Wjk[t][:HID], Wjk[t][HID:], bjk[t],
                                  PAD_SIZES[t])
        pools += [add_p, max_p]
    return _head_call(pools[0], pools[1], pools[2], pools[3], y_base,
                      Wy1, by1, Wy2, by2, G1, gb1, bn1g, bn1b, G2, gb2,
                      bn2g, bn2b, G3, gb3, bn3g, bn3b, G4, gb4)
